# Initial kernel scaffold; baseline (speedup 1.0000x reference)
import jax
import jax.numpy as jnp
from jax.experimental import pallas as pl


def kernel(x, target):
    raise NotImplementedError("write your pallas kernel here")


# speedup vs baseline: 2.5367x; 2.5367x over previous
import functools
import math

import jax
import jax.numpy as jnp
from jax import lax
from jax.experimental import pallas as pl
from jax.experimental.pallas import tpu as pltpu
from jax.experimental.pallas import tpu_sc as plsc

SIZE = 32000
PAD_IDX = 0
N_TOKENS = 4096

_SMOOTH = 0.1
_CONF = 1.0 - _SMOOTH
_EPS = _SMOOTH / (SIZE - 2)
_C_ROW = (SIZE - 2) * _EPS * math.log(_EPS) + _CONF * math.log(_CONF)

L = 16
NC = 2
NS = 16
NW = NC * NS
RPW = N_TOKENS // NW
NCH = RPW // L


_BR = 128


def _rowsum_body(x_ref, o_ref):
    o_ref[...] = jnp.sum(x_ref[...], axis=1, keepdims=True)


def _rowsums(x):
    return pl.pallas_call(
        _rowsum_body,
        grid=(N_TOKENS // _BR,),
        in_specs=[pl.BlockSpec((_BR, SIZE), lambda r: (r, 0))],
        out_specs=pl.BlockSpec((_BR, 1), lambda r: (r, 0)),
        out_shape=jax.ShapeDtypeStruct((N_TOKENS, 1), jnp.float32),
    )(x)


@functools.lru_cache(maxsize=1)
def _build_sc_combine():
    mesh = plsc.VectorSubcoreMesh(
        core_axis_name="c", subcore_axis_name="s",
        num_cores=NC, num_subcores=NS,
    )

    @functools.partial(
        pl.kernel,
        out_type=jax.ShapeDtypeStruct((NC, L), jnp.float32),
        mesh=mesh,
        scratch_types=[
            pltpu.VMEM((RPW,), jnp.int32),
            pltpu.VMEM((RPW,), jnp.float32),
            pltpu.VMEM((RPW,), jnp.int32),
            pltpu.VMEM((RPW,), jnp.int32),
            pltpu.VMEM((RPW,), jnp.float32),
            pltpu.VMEM((RPW,), jnp.float32),
            pltpu.VMEM((L,), jnp.float32),
            pltpu.VMEM_SHARED((NS, L), jnp.float32),
            pltpu.VMEM((NS, L), jnp.float32),
            pltpu.SemaphoreType.DMA,
            pltpu.SemaphoreType.DMA,
        ],
    )
    def sc_combine(xf_hbm, t_hbm, s_hbm, out_hbm,
                   t_v, s_v, it_v, i0_v, gt_v, g0_v, acc_v, shared, all_v,
                   sem_t, sem_0):
        cid = lax.axis_index("c")
        sid = lax.axis_index("s")
        wid = cid * NS + sid
        base = wid * RPW

        pltpu.sync_copy(t_hbm.at[pl.ds(base, RPW)], t_v)
        pltpu.sync_copy(s_hbm.at[pl.ds(base, RPW)], s_v)

        for c in range(NCH):
            t = t_v[pl.ds(c * L, L)]
            rows = (base + c * L) + lax.iota(jnp.int32, L)
            flat0 = rows * SIZE
            it_v[pl.ds(c * L, L)] = flat0 + t
            i0_v[pl.ds(c * L, L)] = flat0

        cp_t = pltpu.async_copy(xf_hbm.at[it_v], gt_v, sem_t)
        cp_0 = pltpu.async_copy(xf_hbm.at[i0_v], g0_v, sem_0)
        cp_t.wait()
        cp_0.wait()

        acc = jnp.zeros((L,), jnp.float32)
        for c in range(NCH):
            t = t_v[pl.ds(c * L, L)]
            s = s_v[pl.ds(c * L, L)]
            g = gt_v[pl.ds(c * L, L)]
            x0 = g0_v[pl.ds(c * L, L)]
            contrib = (jnp.float32(_C_ROW)
                       - jnp.float32(_EPS) * s
                       + jnp.float32(_EPS) * x0
                       + jnp.float32(_EPS - _CONF) * g)
            acc = acc + jnp.where(t != PAD_IDX, contrib, jnp.float32(0.0))

        acc_v[...] = acc
        pltpu.sync_copy(acc_v, shared.at[sid])
        plsc.subcore_barrier()

        @pl.when(sid == 0)
        def _():
            pltpu.sync_copy(shared, all_v)
            tot = jnp.zeros((L,), jnp.float32)
            for w in range(NS):
                tot = tot + all_v[w]
            acc_v[...] = tot
            pltpu.sync_copy(acc_v, out_hbm.at[cid])

    return sc_combine


def _final_body(p_ref, o_ref):
    o_ref[0, 0] = jnp.sum(p_ref[...])


def _final_sum(partials):
    return pl.pallas_call(
        _final_body,
        out_specs=pl.BlockSpec(memory_space=pltpu.SMEM),
        out_shape=jax.ShapeDtypeStruct((1, 1), jnp.float32),
    )(partials)


@jax.jit
def kernel(x, target):
    s = _rowsums(x)
    xf = x.reshape(N_TOKENS * SIZE)
    partials = _build_sc_combine()(xf, target.astype(jnp.int32),
                                   s.reshape(N_TOKENS))
    return _final_sum(partials)[0, 0]
